# baseline (speedup 1.0000x reference)
import jax
import jax.numpy as jnp
from jax.experimental import pallas as pl
from jax.experimental.pallas import tpu as pltpu

B, N, D, HID, S, NH = 2, 384, 768, 256, 64, 8
DH = D // NH
BI = 8
EPS = 1e-5


def _ln(x, g, b):
    m = jnp.mean(x, axis=-1, keepdims=True)
    v = jnp.mean((x - m) ** 2, axis=-1, keepdims=True)
    return (x - m) * jax.lax.rsqrt(v + EPS) * g + b


def _geo_kernel(ci_ref, cj_ref, w1_ref, b1_ref, g1_ref, be1_ref,
                w2_ref, b2_ref, g2_ref, be2_ref, out_ref, h_scratch):
    cj = cj_ref[0]
    cjx = cj[:, 0:1]
    cjy = cj[:, 1:2]
    cjz = cj[:, 2:3]
    b1 = b1_ref[0:1, :]
    g1 = g1_ref[0:1, :]
    be1 = be1_ref[0:1, :]
    for t in range(BI):
        cix = ci_ref[0, t:t + 1, 0:1]
        ciy = ci_ref[0, t:t + 1, 1:2]
        ciz = ci_ref[0, t:t + 1, 2:3]
        dx = cix - cjx
        dy = ciy - cjy
        dz = ciz - cjz
        dist = jnp.sqrt(dx * dx + dy * dy + dz * dz)
        inv = 1.0 / jnp.maximum(dist, 1e-12)
        rx = dx * inv
        ry = dy * inv
        rz = dz * inv
        h = (dist * w1_ref[0:1, :] + rx * w1_ref[1:2, :]
             + ry * w1_ref[2:3, :] + rz * w1_ref[3:4, :]
             + (ry * rz) * w1_ref[6:7, :] + (rx * ry) * w1_ref[7:8, :]
             + (rx * rz) * w1_ref[8:9, :]) + b1
        h = jax.nn.relu(_ln(h, g1, be1))
        h_scratch[t * N:(t + 1) * N, :] = h
    enc = jnp.dot(h_scratch[:], w2_ref[:], preferred_element_type=jnp.float32)
    enc = _ln(enc + b2_ref[0:1, :], g2_ref[0:1, :], be2_ref[0:1, :])
    row = jax.lax.broadcasted_iota(jnp.int32, (BI, BI * N), 0)
    col = jax.lax.broadcasted_iota(jnp.int32, (BI, BI * N), 1)
    sel = jnp.where(col // N == row, 1.0 / N, 0.0)
    out_ref[0] = jnp.dot(sel, enc, preferred_element_type=jnp.float32)


def _agg_attn_kernel(feat_ref, geo_ref, lab_ref,
                     aw1_ref, ab1_ref, ag1_ref, abe1_ref,
                     aw2_ref, ab2_ref, ag2_ref, abe2_ref,
                     wq_ref, bq_ref, wk_ref, bk_ref, wv_ref, bv_ref,
                     wo_ref, bo_ref, out_ref):
    feat = feat_ref[0]
    geo = geo_ref[0]
    lab = lab_ref[0]
    iota_s = jax.lax.broadcasted_iota(jnp.int32, (N, S), 1)
    oh = (lab == iota_s).astype(jnp.float32)
    ones = jnp.ones((N, 1), dtype=jnp.float32)
    cn = (((0,), (0,)), ((), ()))
    cnt = jax.lax.dot_general(oh, ones, cn,
                              preferred_element_type=jnp.float32)
    sum_f = jax.lax.dot_general(oh, feat, cn,
                                preferred_element_type=jnp.float32)
    sum_g = jax.lax.dot_general(oh, geo, cn,
                                preferred_element_type=jnp.float32)
    denom = 1.0 / jnp.maximum(cnt, 1.0)
    mean_f = sum_f * denom
    mean_g = sum_g * denom
    ah = jnp.dot(mean_f, aw1_ref[:], preferred_element_type=jnp.float32)
    ah = jax.nn.relu(_ln(ah + ab1_ref[0:1, :], ag1_ref[0:1, :], abe1_ref[0:1, :]))
    agg = jnp.dot(ah, aw2_ref[:], preferred_element_type=jnp.float32)
    agg = _ln(agg + ab2_ref[0:1, :], ag2_ref[0:1, :], abe2_ref[0:1, :])
    combined = agg + mean_g
    comb_g = jnp.dot(oh, combined, preferred_element_type=jnp.float32)
    cnt_pt = jnp.dot(oh, cnt, preferred_element_type=jnp.float32)
    valid = cnt_pt >= 2.0
    enh = jnp.where(valid, 0.7 * feat + 0.3 * comb_g, feat)
    scale = DH ** -0.5
    acc = jnp.zeros((N, D), dtype=jnp.float32)
    ct = (((1,), (1,)), ((), ()))
    for h in range(NH):
        qh = jnp.dot(enh, wq_ref[h], preferred_element_type=jnp.float32) + bq_ref[h]
        kh = jnp.dot(geo, wk_ref[h], preferred_element_type=jnp.float32) + bk_ref[h]
        vh = jnp.dot(geo, wv_ref[h], preferred_element_type=jnp.float32) + bv_ref[h]
        s = jax.lax.dot_general(qh, kh, ct,
                                preferred_element_type=jnp.float32) * scale
        s = s - jnp.max(s, axis=-1, keepdims=True)
        p = jnp.exp(s)
        p = p * (1.0 / jnp.sum(p, axis=-1, keepdims=True))
        ov = jnp.dot(p, vh, preferred_element_type=jnp.float32)
        acc = acc + jnp.dot(ov, wo_ref[h], preferred_element_type=jnp.float32)
    out_ref[0] = enh + 0.5 * (acc + bo_ref[0:1, :])


def _geo_ctx(coordinates, ge_w1, ge_b1, ge_g1, ge_be1, ge_w2, ge_b2, ge_g2, ge_be2):
    grid = (B, N // BI)
    row2 = lambda x: x.reshape(1, -1)
    return pl.pallas_call(
        _geo_kernel,
        grid=grid,
        in_specs=[
            pl.BlockSpec((1, BI, 3), lambda b, i: (b, i, 0)),
            pl.BlockSpec((1, N, 3), lambda b, i: (b, 0, 0)),
            pl.BlockSpec((9, HID), lambda b, i: (0, 0)),
            pl.BlockSpec((1, HID), lambda b, i: (0, 0)),
            pl.BlockSpec((1, HID), lambda b, i: (0, 0)),
            pl.BlockSpec((1, HID), lambda b, i: (0, 0)),
            pl.BlockSpec((HID, D), lambda b, i: (0, 0)),
            pl.BlockSpec((1, D), lambda b, i: (0, 0)),
            pl.BlockSpec((1, D), lambda b, i: (0, 0)),
            pl.BlockSpec((1, D), lambda b, i: (0, 0)),
        ],
        out_specs=pl.BlockSpec((1, BI, D), lambda b, i: (b, i, 0)),
        out_shape=jax.ShapeDtypeStruct((B, N, D), jnp.float32),
        scratch_shapes=[pltpu.VMEM((BI * N, HID), jnp.float32)],
        compiler_params=pltpu.CompilerParams(
            dimension_semantics=("parallel", "parallel")),
    )(coordinates, coordinates, ge_w1, row2(ge_b1), row2(ge_g1), row2(ge_be1),
      ge_w2, row2(ge_b2), row2(ge_g2), row2(ge_be2))


def kernel(coordinates, features, superpoint_labels,
           ge_w1, ge_b1, ge_g1, ge_be1, ge_w2, ge_b2, ge_g2, ge_be2,
           ag_w1, ag_b1, ag_g1, ag_be1, ag_w2, ag_b2, ag_g2, ag_be2,
           wq, bq, wk, bk, wv, bv, wo, bo):
    geo = _geo_ctx(coordinates, ge_w1, ge_b1, ge_g1, ge_be1,
                   ge_w2, ge_b2, ge_g2, ge_be2)
    lab = superpoint_labels.astype(jnp.int32).reshape(B, N, 1)
    row2 = lambda x: x.reshape(1, -1)
    wq_r = wq.reshape(D, NH, DH).transpose(1, 0, 2)
    wk_r = wk.reshape(D, NH, DH).transpose(1, 0, 2)
    wv_r = wv.reshape(D, NH, DH).transpose(1, 0, 2)
    wo_r = wo.reshape(NH, DH, D)
    bq_r = bq.reshape(NH, 1, DH)
    bk_r = bk.reshape(NH, 1, DH)
    bv_r = bv.reshape(NH, 1, DH)
    full = lambda shp: pl.BlockSpec(shp, lambda b: tuple(0 for _ in shp))
    return pl.pallas_call(
        _agg_attn_kernel,
        grid=(B,),
        in_specs=[
            pl.BlockSpec((1, N, D), lambda b: (b, 0, 0)),
            pl.BlockSpec((1, N, D), lambda b: (b, 0, 0)),
            pl.BlockSpec((1, N, 1), lambda b: (b, 0, 0)),
            full((D, HID)), full((1, HID)), full((1, HID)), full((1, HID)),
            full((HID, D)), full((1, D)), full((1, D)), full((1, D)),
            full((NH, D, DH)), full((NH, 1, DH)),
            full((NH, D, DH)), full((NH, 1, DH)),
            full((NH, D, DH)), full((NH, 1, DH)),
            full((NH, DH, D)), full((1, D)),
        ],
        out_specs=pl.BlockSpec((1, N, D), lambda b: (b, 0, 0)),
        out_shape=jax.ShapeDtypeStruct((B, N, D), jnp.float32),
        compiler_params=pltpu.CompilerParams(
            dimension_semantics=("parallel",)),
    )(features, geo, lab,
      ag_w1, row2(ag_b1), row2(ag_g1), row2(ag_be1),
      ag_w2, row2(ag_b2), row2(ag_g2), row2(ag_be2),
      wq_r, bq_r, wk_r, bk_r, wv_r, bv_r, wo_r, row2(bo))


# speedup vs baseline: 1.9304x; 1.9304x over previous
import jax
import jax.numpy as jnp
from jax.experimental import pallas as pl
from jax.experimental.pallas import tpu as pltpu

B, N, D, HID, S, NH = 2, 384, 768, 256, 64, 8
DH = D // NH
JB = 8
NJ = N // JB
EPS = 1e-5


def _ln(x, g, b):
    m = jnp.mean(x, axis=-1, keepdims=True)
    v = jnp.mean((x - m) ** 2, axis=-1, keepdims=True)
    return (x - m) * jax.lax.rsqrt(v + EPS) * g + b


def _geo_kernel(cj_ref, ci_ref, w1ta_ref, u1_ref, m1_ref, g1b_ref, be1b_ref,
                w2t_ref, u2_ref, mb2_ref, m2_ref, c2_ref, s2_ref,
                b2b_ref, g2n_ref, be2b_ref, out_ref,
                rel_ref, accmu_ref, accs_ref):
    j_blk = pl.program_id(1)

    @pl.when(j_blk == 0)
    def _():
        out_ref[0] = jnp.zeros((D, N), dtype=jnp.float32)
        accmu_ref[...] = jnp.zeros((1, N), dtype=jnp.float32)
        accs_ref[...] = jnp.zeros((1, N), dtype=jnp.float32)

    cix = ci_ref[0, 0:1, :]
    ciy = ci_ref[0, 1:2, :]
    ciz = ci_ref[0, 2:3, :]
    ones_row = jnp.ones((1, N), dtype=jnp.float32)
    for t in range(JB):
        cjx = cj_ref[0, t:t + 1, 0:1]
        cjy = cj_ref[0, t:t + 1, 1:2]
        cjz = cj_ref[0, t:t + 1, 2:3]
        dx = cix - cjx
        dy = ciy - cjy
        dz = ciz - cjz
        dist = jnp.sqrt(dx * dx + dy * dy + dz * dz)
        inv = 1.0 / jnp.maximum(dist, 1e-12)
        rx = dx * inv
        ry = dy * inv
        rz = dz * inv
        rel_ref[0:1, :] = dist
        rel_ref[1:2, :] = rx
        rel_ref[2:3, :] = ry
        rel_ref[3:4, :] = rz
        rel_ref[4:5, :] = ry * rz
        rel_ref[5:6, :] = rx * ry
        rel_ref[6:7, :] = rx * rz
        rel_ref[7:8, :] = ones_row
        rel = rel_ref[...]
        hpre = jnp.dot(w1ta_ref[...], rel, preferred_element_type=jnp.float32)
        mu1 = jnp.dot(u1_ref[...], rel, preferred_element_type=jnp.float32)
        q1 = jnp.dot(m1_ref[...], rel, preferred_element_type=jnp.float32)
        ms1 = jnp.sum(rel * q1, axis=0, keepdims=True)
        rs1 = jax.lax.rsqrt(jnp.maximum(ms1 - mu1 * mu1, 0.0) + EPS)
        h = jax.nn.relu((hpre - mu1) * rs1 * g1b_ref[...] + be1b_ref[...])
        e_raw = jnp.dot(w2t_ref[...], h, preferred_element_type=jnp.float32)
        mu2 = jnp.dot(u2_ref[...], h,
                      preferred_element_type=jnp.float32) + mb2_ref[...]
        q2 = jnp.dot(m2_ref[...], h, preferred_element_type=jnp.float32)
        c2h = jnp.dot(c2_ref[...], h, preferred_element_type=jnp.float32)
        ms2 = jnp.sum(h * q2, axis=0, keepdims=True) + 2.0 * c2h + s2_ref[...]
        rs2 = jax.lax.rsqrt(jnp.maximum(ms2 - mu2 * mu2, 0.0) + EPS)
        out_ref[0] += e_raw * rs2
        accmu_ref[...] += mu2 * rs2
        accs_ref[...] += rs2

    @pl.when(j_blk == NJ - 1)
    def _():
        a = out_ref[0] - accmu_ref[...] + b2b_ref[...] * accs_ref[...]
        out_ref[0] = a * g2n_ref[...] + be2b_ref[...]


def _agg_attn_kernel(feat_ref, geo_ref, lab_ref,
                     aw1_ref, ab1_ref, ag1_ref, abe1_ref,
                     aw2_ref, ab2_ref, ag2_ref, abe2_ref,
                     wq_ref, bq_ref, wk_ref, bk_ref, wv_ref, bv_ref,
                     wo_ref, bo_ref, out_ref):
    feat = feat_ref[0]
    geo = geo_ref[0]
    lab = lab_ref[0]
    iota_s = jax.lax.broadcasted_iota(jnp.int32, (N, S), 1)
    oh = (lab == iota_s).astype(jnp.float32)
    ones = jnp.ones((N, 1), dtype=jnp.float32)
    cn = (((0,), (0,)), ((), ()))
    cnt = jax.lax.dot_general(oh, ones, cn,
                              preferred_element_type=jnp.float32)
    sum_f = jax.lax.dot_general(oh, feat, cn,
                                preferred_element_type=jnp.float32)
    sum_g = jax.lax.dot_general(oh, geo, cn,
                                preferred_element_type=jnp.float32)
    denom = 1.0 / jnp.maximum(cnt, 1.0)
    mean_f = sum_f * denom
    mean_g = sum_g * denom
    ah = jnp.dot(mean_f, aw1_ref[...], preferred_element_type=jnp.float32)
    ah = jax.nn.relu(_ln(ah + ab1_ref[0:1, :], ag1_ref[0:1, :], abe1_ref[0:1, :]))
    agg = jnp.dot(ah, aw2_ref[...], preferred_element_type=jnp.float32)
    agg = _ln(agg + ab2_ref[0:1, :], ag2_ref[0:1, :], abe2_ref[0:1, :])
    combined = agg + mean_g
    comb_g = jnp.dot(oh, combined, preferred_element_type=jnp.float32)
    cnt_pt = jnp.dot(oh, cnt, preferred_element_type=jnp.float32)
    valid = cnt_pt >= 2.0
    enh = jnp.where(valid, 0.7 * feat + 0.3 * comb_g, feat)
    scale = DH ** -0.5
    acc = jnp.zeros((N, D), dtype=jnp.float32)
    ct = (((1,), (1,)), ((), ()))
    for h in range(NH):
        qh = jnp.dot(enh, wq_ref[h], preferred_element_type=jnp.float32) + bq_ref[h]
        kh = jnp.dot(geo, wk_ref[h], preferred_element_type=jnp.float32) + bk_ref[h]
        vh = jnp.dot(geo, wv_ref[h], preferred_element_type=jnp.float32) + bv_ref[h]
        s = jax.lax.dot_general(qh, kh, ct,
                                preferred_element_type=jnp.float32) * scale
        s = s - jnp.max(s, axis=-1, keepdims=True)
        p = jnp.exp(s)
        p = p * (1.0 / jnp.sum(p, axis=-1, keepdims=True))
        ov = jnp.dot(p, vh, preferred_element_type=jnp.float32)
        acc = acc + jnp.dot(ov, wo_ref[h], preferred_element_type=jnp.float32)
    out_ref[0] = enh + 0.5 * (acc + bo_ref[0:1, :])


def _geo_ctx(coordinates, ge_w1, ge_b1, ge_g1, ge_be1, ge_w2, ge_b2, ge_g2, ge_be2):
    coords_t = coordinates.transpose(0, 2, 1)
    w1ta = jnp.stack([ge_w1[0], ge_w1[1], ge_w1[2], ge_w1[3],
                      ge_w1[6], ge_w1[7], ge_w1[8], ge_b1], axis=1)
    u1 = jnp.mean(w1ta, axis=0, keepdims=True)
    m1 = (w1ta.T @ w1ta) / HID
    g1b = jnp.broadcast_to(ge_g1[:, None], (HID, N))
    be1b = jnp.broadcast_to(ge_be1[:, None], (HID, N))
    w2t = ge_w2.T
    u2 = jnp.mean(w2t, axis=0, keepdims=True)
    mb2 = jnp.mean(ge_b2).reshape(1, 1)
    m2 = (ge_w2 @ w2t) / D
    c2 = (ge_b2[None, :] @ w2t) / D
    s2 = jnp.mean(ge_b2 * ge_b2).reshape(1, 1)
    b2b = jnp.broadcast_to(ge_b2[:, None], (D, N))
    g2n = jnp.broadcast_to(ge_g2[:, None] / N, (D, N))
    be2b = jnp.broadcast_to(ge_be2[:, None], (D, N))
    full = lambda shp: pl.BlockSpec(shp, lambda b, j: tuple(0 for _ in shp))
    geo_t = pl.pallas_call(
        _geo_kernel,
        grid=(B, NJ),
        in_specs=[
            pl.BlockSpec((1, JB, 3), lambda b, j: (b, j, 0)),
            pl.BlockSpec((1, 3, N), lambda b, j: (b, 0, 0)),
            full((HID, 8)), full((1, 8)), full((8, 8)),
            full((HID, N)), full((HID, N)),
            full((D, HID)), full((1, HID)), full((1, 1)),
            full((HID, HID)), full((1, HID)), full((1, 1)),
            full((D, N)), full((D, N)), full((D, N)),
        ],
        out_specs=pl.BlockSpec((1, D, N), lambda b, j: (b, 0, 0)),
        out_shape=jax.ShapeDtypeStruct((B, D, N), jnp.float32),
        scratch_shapes=[pltpu.VMEM((8, N), jnp.float32),
                        pltpu.VMEM((1, N), jnp.float32),
                        pltpu.VMEM((1, N), jnp.float32)],
        compiler_params=pltpu.CompilerParams(
            dimension_semantics=("arbitrary", "arbitrary")),
    )(coordinates, coords_t, w1ta, u1, m1, g1b, be1b,
      w2t, u2, mb2, m2, c2, s2, b2b, g2n, be2b)
    return geo_t.transpose(0, 2, 1)


def kernel(coordinates, features, superpoint_labels,
           ge_w1, ge_b1, ge_g1, ge_be1, ge_w2, ge_b2, ge_g2, ge_be2,
           ag_w1, ag_b1, ag_g1, ag_be1, ag_w2, ag_b2, ag_g2, ag_be2,
           wq, bq, wk, bk, wv, bv, wo, bo):
    geo = _geo_ctx(coordinates, ge_w1, ge_b1, ge_g1, ge_be1,
                   ge_w2, ge_b2, ge_g2, ge_be2)
    lab = superpoint_labels.astype(jnp.int32).reshape(B, N, 1)
    row2 = lambda x: x.reshape(1, -1)
    wq_r = wq.reshape(D, NH, DH).transpose(1, 0, 2)
    wk_r = wk.reshape(D, NH, DH).transpose(1, 0, 2)
    wv_r = wv.reshape(D, NH, DH).transpose(1, 0, 2)
    wo_r = wo.reshape(NH, DH, D)
    bq_r = bq.reshape(NH, 1, DH)
    bk_r = bk.reshape(NH, 1, DH)
    bv_r = bv.reshape(NH, 1, DH)
    full = lambda shp: pl.BlockSpec(shp, lambda b: tuple(0 for _ in shp))
    return pl.pallas_call(
        _agg_attn_kernel,
        grid=(B,),
        in_specs=[
            pl.BlockSpec((1, N, D), lambda b: (b, 0, 0)),
            pl.BlockSpec((1, N, D), lambda b: (b, 0, 0)),
            pl.BlockSpec((1, N, 1), lambda b: (b, 0, 0)),
            full((D, HID)), full((1, HID)), full((1, HID)), full((1, HID)),
            full((HID, D)), full((1, D)), full((1, D)), full((1, D)),
            full((NH, D, DH)), full((NH, 1, DH)),
            full((NH, D, DH)), full((NH, 1, DH)),
            full((NH, D, DH)), full((NH, 1, DH)),
            full((NH, DH, D)), full((1, D)),
        ],
        out_specs=pl.BlockSpec((1, N, D), lambda b: (b, 0, 0)),
        out_shape=jax.ShapeDtypeStruct((B, N, D), jnp.float32),
        compiler_params=pltpu.CompilerParams(
            dimension_semantics=("arbitrary",)),
    )(features, geo, lab,
      ag_w1, row2(ag_b1), row2(ag_g1), row2(ag_be1),
      ag_w2, row2(ag_b2), row2(ag_g2), row2(ag_be2),
      wq_r, bq_r, wk_r, bk_r, wv_r, bv_r, wo_r, row2(bo))


# speedup vs baseline: 2.9104x; 1.5077x over previous
import jax
import jax.numpy as jnp
from jax.experimental import pallas as pl
from jax.experimental.pallas import tpu as pltpu

B, N, D, HID, S, NH = 2, 384, 768, 256, 64, 8
DH = D // NH
JB = 8
NJ = N // JB
EPS = 1e-5


def _ln(x, g, b):
    m = jnp.mean(x, axis=-1, keepdims=True)
    v = jnp.mean((x - m) ** 2, axis=-1, keepdims=True)
    return (x - m) * jax.lax.rsqrt(v + EPS) * g + b


def _geo_kernel(cj_ref, ci_ref, w1g_ref, u1_ref, m1_ref, g1b_ref, be1b_ref,
                w2t_ref, b2r_ref, b2c_ref, g2n_ref, be2c_ref, out_ref,
                rel_ref, hacc_ref, accmu_ref, accs_ref,
                m2b_ref, u2b_ref, c2b_ref, sc_ref):
    j_blk = pl.program_id(1)

    @pl.when(j_blk == 0)
    def _():
        w2t = w2t_ref[...]
        gram = jax.lax.dot_general(
            w2t, w2t, (((0,), (0,)), ((), ())),
            preferred_element_type=jnp.float32)
        m2b_ref[...] = (gram * (1.0 / D)).astype(jnp.bfloat16)
        u2b_ref[...] = jnp.mean(w2t, axis=0, keepdims=True).astype(jnp.bfloat16)
        b2r = b2r_ref[...]
        c2 = jnp.dot(b2r, w2t, preferred_element_type=jnp.float32) * (1.0 / D)
        c2b_ref[...] = c2.astype(jnp.bfloat16)
        sc_ref[0:1, 0:1] = jnp.mean(b2r, axis=1, keepdims=True)
        sc_ref[0:1, 1:2] = jnp.mean(b2r * b2r, axis=1, keepdims=True)
        hacc_ref[...] = jnp.zeros((HID, N), dtype=jnp.float32)
        accmu_ref[...] = jnp.zeros((1, N), dtype=jnp.float32)
        accs_ref[...] = jnp.zeros((1, N), dtype=jnp.float32)

    cix = ci_ref[0, 0:1, :]
    ciy = ci_ref[0, 1:2, :]
    ciz = ci_ref[0, 2:3, :]
    ones_row = jnp.ones((1, N), dtype=jnp.float32)
    mb2 = sc_ref[0:1, 0:1]
    s2 = sc_ref[0:1, 1:2]
    for t in range(JB):
        cjx = cj_ref[0, t:t + 1, 0:1]
        cjy = cj_ref[0, t:t + 1, 1:2]
        cjz = cj_ref[0, t:t + 1, 2:3]
        dx = cix - cjx
        dy = ciy - cjy
        dz = ciz - cjz
        dist = jnp.sqrt(dx * dx + dy * dy + dz * dz)
        inv = 1.0 / jnp.maximum(dist, 1e-12)
        rx = dx * inv
        ry = dy * inv
        rz = dz * inv
        rel_ref[0:1, :] = dist
        rel_ref[1:2, :] = rx
        rel_ref[2:3, :] = ry
        rel_ref[3:4, :] = rz
        rel_ref[4:5, :] = ry * rz
        rel_ref[5:6, :] = rx * ry
        rel_ref[6:7, :] = rx * rz
        rel_ref[7:8, :] = ones_row
        rel = rel_ref[...]
        mu1 = jnp.dot(u1_ref[...], rel, preferred_element_type=jnp.float32)
        q1 = jnp.dot(m1_ref[...], rel, preferred_element_type=jnp.float32)
        ms1 = jnp.sum(rel * q1, axis=0, keepdims=True)
        rs1 = jax.lax.rsqrt(jnp.maximum(ms1 - mu1 * mu1, 0.0) + EPS)
        rels = rel * rs1
        hp = jnp.dot(w1g_ref[...], rels, preferred_element_type=jnp.float32)
        adj = g1b_ref[...] * (mu1 * rs1) - be1b_ref[...]
        h = jax.nn.relu(hp - adj)
        hb = h.astype(jnp.bfloat16)
        mu2 = jnp.dot(u2b_ref[...], hb,
                      preferred_element_type=jnp.float32) + mb2
        q2 = jnp.dot(m2b_ref[...], hb, preferred_element_type=jnp.float32)
        c2h = jnp.dot(c2b_ref[...], hb, preferred_element_type=jnp.float32)
        ms2 = jnp.sum(h * q2, axis=0, keepdims=True) + 2.0 * c2h + s2
        rs2 = jax.lax.rsqrt(jnp.maximum(ms2 - mu2 * mu2, 0.0) + EPS)
        hacc_ref[...] += h * rs2
        accmu_ref[...] += mu2 * rs2
        accs_ref[...] += rs2

    @pl.when(j_blk == NJ - 1)
    def _():
        a = jnp.dot(w2t_ref[...], hacc_ref[...],
                    preferred_element_type=jnp.float32)
        a = a - accmu_ref[...] + b2c_ref[...] * accs_ref[...]
        out_ref[0] = a * g2n_ref[...] + be2c_ref[...]


def _agg_attn_kernel(feat_ref, geo_ref, lab_ref,
                     aw1_ref, ab1_ref, ag1_ref, abe1_ref,
                     aw2_ref, ab2_ref, ag2_ref, abe2_ref,
                     wq_ref, bq_ref, wk_ref, bk_ref, wv_ref, bv_ref,
                     wo_ref, bo_ref, out_ref):
    feat = feat_ref[0]
    geo = geo_ref[0]
    lab = lab_ref[0]
    iota_s = jax.lax.broadcasted_iota(jnp.int32, (N, S), 1)
    oh = (lab == iota_s).astype(jnp.float32)
    ones = jnp.ones((N, 1), dtype=jnp.float32)
    cn = (((0,), (0,)), ((), ()))
    cnt = jax.lax.dot_general(oh, ones, cn,
                              preferred_element_type=jnp.float32)
    sum_f = jax.lax.dot_general(oh, feat, cn,
                                preferred_element_type=jnp.float32)
    sum_g = jax.lax.dot_general(oh, geo, cn,
                                preferred_element_type=jnp.float32)
    denom = 1.0 / jnp.maximum(cnt, 1.0)
    mean_f = sum_f * denom
    mean_g = sum_g * denom
    ah = jnp.dot(mean_f, aw1_ref[...], preferred_element_type=jnp.float32)
    ah = jax.nn.relu(_ln(ah + ab1_ref[0:1, :], ag1_ref[0:1, :], abe1_ref[0:1, :]))
    agg = jnp.dot(ah, aw2_ref[...], preferred_element_type=jnp.float32)
    agg = _ln(agg + ab2_ref[0:1, :], ag2_ref[0:1, :], abe2_ref[0:1, :])
    combined = agg + mean_g
    comb_g = jnp.dot(oh, combined, preferred_element_type=jnp.float32)
    cnt_pt = jnp.dot(oh, cnt, preferred_element_type=jnp.float32)
    valid = cnt_pt >= 2.0
    enh = jnp.where(valid, 0.7 * feat + 0.3 * comb_g, feat)
    scale = DH ** -0.5
    acc = jnp.zeros((N, D), dtype=jnp.float32)
    ct = (((1,), (1,)), ((), ()))
    for h in range(NH):
        qh = jnp.dot(enh, wq_ref[h], preferred_element_type=jnp.float32) + bq_ref[h]
        kh = jnp.dot(geo, wk_ref[h], preferred_element_type=jnp.float32) + bk_ref[h]
        vh = jnp.dot(geo, wv_ref[h], preferred_element_type=jnp.float32) + bv_ref[h]
        s = jax.lax.dot_general(qh, kh, ct,
                                preferred_element_type=jnp.float32) * scale
        s = s - jnp.max(s, axis=-1, keepdims=True)
        p = jnp.exp(s)
        p = p * (1.0 / jnp.sum(p, axis=-1, keepdims=True))
        ov = jnp.dot(p, vh, preferred_element_type=jnp.float32)
        acc = acc + jnp.dot(ov, wo_ref[h], preferred_element_type=jnp.float32)
    out_ref[0] = enh + 0.5 * (acc + bo_ref[0:1, :])


def _geo_ctx(coordinates, ge_w1, ge_b1, ge_g1, ge_be1, ge_w2, ge_b2, ge_g2, ge_be2):
    coords_t = coordinates.transpose(0, 2, 1)
    w1ta = jnp.stack([ge_w1[0], ge_w1[1], ge_w1[2], ge_w1[3],
                      ge_w1[6], ge_w1[7], ge_w1[8], ge_b1], axis=1)
    u1 = jnp.mean(w1ta, axis=0, keepdims=True)
    m1 = (w1ta.T @ w1ta) / HID
    w1g = w1ta * ge_g1[:, None]
    g1b = jnp.broadcast_to(ge_g1[:, None], (HID, N))
    be1b = jnp.broadcast_to(ge_be1[:, None], (HID, N))
    w2t = ge_w2.T
    full = lambda shp: pl.BlockSpec(shp, lambda b, j: tuple(0 for _ in shp))
    geo_t = pl.pallas_call(
        _geo_kernel,
        grid=(B, NJ),
        in_specs=[
            pl.BlockSpec((1, JB, 3), lambda b, j: (b, j, 0)),
            pl.BlockSpec((1, 3, N), lambda b, j: (b, 0, 0)),
            full((HID, 8)), full((1, 8)), full((8, 8)),
            full((HID, N)), full((HID, N)),
            full((D, HID)), full((1, D)),
            full((D, 1)), full((D, 1)), full((D, 1)),
        ],
        out_specs=pl.BlockSpec((1, D, N), lambda b, j: (b, 0, 0)),
        out_shape=jax.ShapeDtypeStruct((B, D, N), jnp.float32),
        scratch_shapes=[pltpu.VMEM((8, N), jnp.float32),
                        pltpu.VMEM((HID, N), jnp.float32),
                        pltpu.VMEM((1, N), jnp.float32),
                        pltpu.VMEM((1, N), jnp.float32),
                        pltpu.VMEM((HID, HID), jnp.bfloat16),
                        pltpu.VMEM((1, HID), jnp.bfloat16),
                        pltpu.VMEM((1, HID), jnp.bfloat16),
                        pltpu.VMEM((1, 128), jnp.float32)],
        compiler_params=pltpu.CompilerParams(
            dimension_semantics=("arbitrary", "arbitrary")),
    )(coordinates, coords_t, w1g, u1, m1, g1b, be1b,
      w2t, ge_b2.reshape(1, D),
      ge_b2.reshape(D, 1), (ge_g2 / N).reshape(D, 1), ge_be2.reshape(D, 1))
    return geo_t.transpose(0, 2, 1)


def kernel(coordinates, features, superpoint_labels,
           ge_w1, ge_b1, ge_g1, ge_be1, ge_w2, ge_b2, ge_g2, ge_be2,
           ag_w1, ag_b1, ag_g1, ag_be1, ag_w2, ag_b2, ag_g2, ag_be2,
           wq, bq, wk, bk, wv, bv, wo, bo):
    geo = _geo_ctx(coordinates, ge_w1, ge_b1, ge_g1, ge_be1,
                   ge_w2, ge_b2, ge_g2, ge_be2)
    lab = superpoint_labels.astype(jnp.int32).reshape(B, N, 1)
    row2 = lambda x: x.reshape(1, -1)
    wq_r = wq.reshape(D, NH, DH).transpose(1, 0, 2)
    wk_r = wk.reshape(D, NH, DH).transpose(1, 0, 2)
    wv_r = wv.reshape(D, NH, DH).transpose(1, 0, 2)
    wo_r = wo.reshape(NH, DH, D)
    bq_r = bq.reshape(NH, 1, DH)
    bk_r = bk.reshape(NH, 1, DH)
    bv_r = bv.reshape(NH, 1, DH)
    full = lambda shp: pl.BlockSpec(shp, lambda b: tuple(0 for _ in shp))
    return pl.pallas_call(
        _agg_attn_kernel,
        grid=(B,),
        in_specs=[
            pl.BlockSpec((1, N, D), lambda b: (b, 0, 0)),
            pl.BlockSpec((1, N, D), lambda b: (b, 0, 0)),
            pl.BlockSpec((1, N, 1), lambda b: (b, 0, 0)),
            full((D, HID)), full((1, HID)), full((1, HID)), full((1, HID)),
            full((HID, D)), full((1, D)), full((1, D)), full((1, D)),
            full((NH, D, DH)), full((NH, 1, DH)),
            full((NH, D, DH)), full((NH, 1, DH)),
            full((NH, D, DH)), full((NH, 1, DH)),
            full((NH, DH, D)), full((1, D)),
        ],
        out_specs=pl.BlockSpec((1, N, D), lambda b: (b, 0, 0)),
        out_shape=jax.ShapeDtypeStruct((B, N, D), jnp.float32),
        compiler_params=pltpu.CompilerParams(
            dimension_semantics=("arbitrary",)),
    )(features, geo, lab,
      ag_w1, row2(ag_b1), row2(ag_g1), row2(ag_be1),
      ag_w2, row2(ag_b2), row2(ag_g2), row2(ag_be2),
      wq_r, bq_r, wk_r, bk_r, wv_r, bv_r, wo_r, row2(bo))


# speedup vs baseline: 3.1153x; 1.0704x over previous
import jax
import jax.numpy as jnp
from jax.experimental import pallas as pl
from jax.experimental.pallas import tpu as pltpu

B, N, D, HID, S, NH = 2, 384, 768, 256, 64, 8
DH = D // NH
JB = 8
NJ = N // JB
EPS = 1e-5


def _ln(x, g, b):
    m = jnp.mean(x, axis=-1, keepdims=True)
    v = jnp.mean((x - m) ** 2, axis=-1, keepdims=True)
    return (x - m) * jax.lax.rsqrt(v + EPS) * g + b


def _geo_kernel(cj_ref, ci_ref, w1g_ref, u1_ref, m1_ref, g1c_ref, be1c_ref,
                w2t_ref, b2r_ref, b2c_ref, g2n_ref, be2c_ref, out_ref,
                rel_ref, hacc_ref, accmu_ref, accs_ref,
                m2b_ref, u2b_ref, c2b_ref, sc_ref, g1b_ref, be1b_ref):
    j_blk = pl.program_id(1)

    @pl.when(j_blk == 0)
    def _():
        g1b_ref[...] = jnp.broadcast_to(g1c_ref[...], (HID, N))
        be1b_ref[...] = jnp.broadcast_to(be1c_ref[...], (HID, N))
        w2t = w2t_ref[...]
        gram = jax.lax.dot_general(
            w2t, w2t, (((0,), (0,)), ((), ())),
            preferred_element_type=jnp.float32)
        m2b_ref[...] = (gram * (1.0 / D)).astype(jnp.bfloat16)
        u2b_ref[...] = jnp.mean(w2t, axis=0, keepdims=True).astype(jnp.bfloat16)
        b2r = b2r_ref[...]
        c2 = jnp.dot(b2r, w2t, preferred_element_type=jnp.float32) * (1.0 / D)
        c2b_ref[...] = c2.astype(jnp.bfloat16)
        sc_ref[0:1, 0:1] = jnp.mean(b2r, axis=1, keepdims=True)
        sc_ref[0:1, 1:2] = jnp.mean(b2r * b2r, axis=1, keepdims=True)
        hacc_ref[...] = jnp.zeros((HID, N), dtype=jnp.float32)
        accmu_ref[...] = jnp.zeros((1, N), dtype=jnp.float32)
        accs_ref[...] = jnp.zeros((1, N), dtype=jnp.float32)

    cix = ci_ref[0, 0:1, :]
    ciy = ci_ref[0, 1:2, :]
    ciz = ci_ref[0, 2:3, :]
    ones_row = jnp.ones((1, N), dtype=jnp.float32)
    mb2 = sc_ref[0:1, 0:1]
    s2 = sc_ref[0:1, 1:2]
    for t in range(JB):
        cjx = cj_ref[0, t:t + 1, 0:1]
        cjy = cj_ref[0, t:t + 1, 1:2]
        cjz = cj_ref[0, t:t + 1, 2:3]
        dx = cix - cjx
        dy = ciy - cjy
        dz = ciz - cjz
        dist = jnp.sqrt(dx * dx + dy * dy + dz * dz)
        inv = 1.0 / jnp.maximum(dist, 1e-12)
        rx = dx * inv
        ry = dy * inv
        rz = dz * inv
        rel_ref[0:1, :] = dist
        rel_ref[1:2, :] = rx
        rel_ref[2:3, :] = ry
        rel_ref[3:4, :] = rz
        rel_ref[4:5, :] = ry * rz
        rel_ref[5:6, :] = rx * ry
        rel_ref[6:7, :] = rx * rz
        rel_ref[7:8, :] = ones_row
        rel = rel_ref[...]
        mu1 = jnp.dot(u1_ref[...], rel, preferred_element_type=jnp.float32)
        q1 = jnp.dot(m1_ref[...], rel, preferred_element_type=jnp.float32)
        ms1 = jnp.sum(rel * q1, axis=0, keepdims=True)
        rs1 = jax.lax.rsqrt(jnp.maximum(ms1 - mu1 * mu1, 0.0) + EPS)
        rels = rel * rs1
        hp = jnp.dot(w1g_ref[...], rels, preferred_element_type=jnp.float32)
        adj = g1b_ref[...] * (mu1 * rs1) - be1b_ref[...]
        h = jax.nn.relu(hp - adj)
        hb = h.astype(jnp.bfloat16)
        mu2 = jnp.dot(u2b_ref[...], hb,
                      preferred_element_type=jnp.float32) + mb2
        q2 = jnp.dot(m2b_ref[...], hb, preferred_element_type=jnp.float32)
        c2h = jnp.dot(c2b_ref[...], hb, preferred_element_type=jnp.float32)
        ms2 = jnp.sum(h * q2, axis=0, keepdims=True) + 2.0 * c2h + s2
        rs2 = jax.lax.rsqrt(jnp.maximum(ms2 - mu2 * mu2, 0.0) + EPS)
        hacc_ref[...] += h * rs2
        accmu_ref[...] += mu2 * rs2
        accs_ref[...] += rs2

    @pl.when(j_blk == NJ - 1)
    def _():
        a = jnp.dot(w2t_ref[...], hacc_ref[...],
                    preferred_element_type=jnp.float32)
        a = a - accmu_ref[...] + b2c_ref[...] * accs_ref[...]
        out_ref[0] = a * g2n_ref[...] + be2c_ref[...]


def _agg_attn_kernel(feat_ref, geo_ref, lab_ref,
                     aw1_ref, ab1_ref, ag1_ref, abe1_ref,
                     aw2_ref, ab2_ref, ag2_ref, abe2_ref,
                     wq_ref, bq_ref, wk_ref, bk_ref, wv_ref, bv_ref,
                     wo_ref, bo_ref, out_ref):
    feat = feat_ref[0]
    geo = geo_ref[0]
    lab = lab_ref[0]
    iota_s = jax.lax.broadcasted_iota(jnp.int32, (N, S), 1)
    oh = (lab == iota_s).astype(jnp.float32)
    ones = jnp.ones((N, 1), dtype=jnp.float32)
    cn = (((0,), (0,)), ((), ()))
    cnt = jax.lax.dot_general(oh, ones, cn,
                              preferred_element_type=jnp.float32)
    sum_f = jax.lax.dot_general(oh, feat, cn,
                                preferred_element_type=jnp.float32)
    sum_g = jax.lax.dot_general(oh, geo, cn,
                                preferred_element_type=jnp.float32)
    denom = 1.0 / jnp.maximum(cnt, 1.0)
    mean_f = sum_f * denom
    mean_g = sum_g * denom
    ah = jnp.dot(mean_f, aw1_ref[...], preferred_element_type=jnp.float32)
    ah = jax.nn.relu(_ln(ah + ab1_ref[0:1, :], ag1_ref[0:1, :], abe1_ref[0:1, :]))
    agg = jnp.dot(ah, aw2_ref[...], preferred_element_type=jnp.float32)
    agg = _ln(agg + ab2_ref[0:1, :], ag2_ref[0:1, :], abe2_ref[0:1, :])
    combined = agg + mean_g
    comb_g = jnp.dot(oh, combined, preferred_element_type=jnp.float32)
    cnt_pt = jnp.dot(oh, cnt, preferred_element_type=jnp.float32)
    valid = cnt_pt >= 2.0
    enh = jnp.where(valid, 0.7 * feat + 0.3 * comb_g, feat)
    scale = DH ** -0.5
    enh_b = enh.astype(jnp.bfloat16)
    geo_b = geo.astype(jnp.bfloat16)
    q = (jnp.dot(enh_b, wq_ref[...].astype(jnp.bfloat16),
                 preferred_element_type=jnp.float32)
         + bq_ref[...]).astype(jnp.bfloat16)
    k = (jnp.dot(geo_b, wk_ref[...].astype(jnp.bfloat16),
                 preferred_element_type=jnp.float32)
         + bk_ref[...]).astype(jnp.bfloat16)
    v = jnp.dot(geo_b, wv_ref[...].astype(jnp.bfloat16),
                preferred_element_type=jnp.float32) + bv_ref[...]
    acc = jnp.zeros((N, D), dtype=jnp.float32)
    ct = (((1,), (1,)), ((), ()))
    for h in range(NH):
        qh = q[:, h * DH:(h + 1) * DH]
        kh = k[:, h * DH:(h + 1) * DH]
        vh = v[:, h * DH:(h + 1) * DH]
        s = jax.lax.dot_general(qh, kh, ct,
                                preferred_element_type=jnp.float32) * scale
        s = s - jnp.max(s, axis=-1, keepdims=True)
        p = jnp.exp(s)
        p = p * (1.0 / jnp.sum(p, axis=-1, keepdims=True))
        ov = jnp.dot(p, vh, preferred_element_type=jnp.float32)
        acc = acc + jnp.dot(ov, wo_ref[h], preferred_element_type=jnp.float32)
    out_ref[0] = enh + 0.5 * (acc + bo_ref[0:1, :])


def _geo_ctx(coordinates, ge_w1, ge_b1, ge_g1, ge_be1, ge_w2, ge_b2, ge_g2, ge_be2):
    coords_t = coordinates.transpose(0, 2, 1)
    w1ta = jnp.stack([ge_w1[0], ge_w1[1], ge_w1[2], ge_w1[3],
                      ge_w1[6], ge_w1[7], ge_w1[8], ge_b1], axis=1)
    u1 = jnp.mean(w1ta, axis=0, keepdims=True)
    m1 = (w1ta.T @ w1ta) / HID
    w1g = w1ta * ge_g1[:, None]
    w2t = ge_w2.T
    full = lambda shp: pl.BlockSpec(shp, lambda b, j: tuple(0 for _ in shp))
    geo_t = pl.pallas_call(
        _geo_kernel,
        grid=(B, NJ),
        in_specs=[
            pl.BlockSpec((1, JB, 3), lambda b, j: (b, j, 0)),
            pl.BlockSpec((1, 3, N), lambda b, j: (b, 0, 0)),
            full((HID, 8)), full((1, 8)), full((8, 8)),
            full((HID, 1)), full((HID, 1)),
            full((D, HID)), full((1, D)),
            full((D, 1)), full((D, 1)), full((D, 1)),
        ],
        out_specs=pl.BlockSpec((1, D, N), lambda b, j: (b, 0, 0)),
        out_shape=jax.ShapeDtypeStruct((B, D, N), jnp.float32),
        scratch_shapes=[pltpu.VMEM((8, N), jnp.float32),
                        pltpu.VMEM((HID, N), jnp.float32),
                        pltpu.VMEM((1, N), jnp.float32),
                        pltpu.VMEM((1, N), jnp.float32),
                        pltpu.VMEM((HID, HID), jnp.bfloat16),
                        pltpu.VMEM((1, HID), jnp.bfloat16),
                        pltpu.VMEM((1, HID), jnp.bfloat16),
                        pltpu.VMEM((1, 128), jnp.float32),
                        pltpu.VMEM((HID, N), jnp.float32),
                        pltpu.VMEM((HID, N), jnp.float32)],
        compiler_params=pltpu.CompilerParams(
            dimension_semantics=("arbitrary", "arbitrary")),
    )(coordinates, coords_t, w1g, u1, m1,
      ge_g1.reshape(HID, 1), ge_be1.reshape(HID, 1),
      w2t, ge_b2.reshape(1, D),
      ge_b2.reshape(D, 1), (ge_g2 / N).reshape(D, 1), ge_be2.reshape(D, 1))
    return geo_t.transpose(0, 2, 1)


def kernel(coordinates, features, superpoint_labels,
           ge_w1, ge_b1, ge_g1, ge_be1, ge_w2, ge_b2, ge_g2, ge_be2,
           ag_w1, ag_b1, ag_g1, ag_be1, ag_w2, ag_b2, ag_g2, ag_be2,
           wq, bq, wk, bk, wv, bv, wo, bo):
    geo = _geo_ctx(coordinates, ge_w1, ge_b1, ge_g1, ge_be1,
                   ge_w2, ge_b2, ge_g2, ge_be2)
    lab = superpoint_labels.astype(jnp.int32).reshape(B, N, 1)
    row2 = lambda x: x.reshape(1, -1)
    wo_r = wo.reshape(NH, DH, D)
    full = lambda shp: pl.BlockSpec(shp, lambda b: tuple(0 for _ in shp))
    return pl.pallas_call(
        _agg_attn_kernel,
        grid=(B,),
        in_specs=[
            pl.BlockSpec((1, N, D), lambda b: (b, 0, 0)),
            pl.BlockSpec((1, N, D), lambda b: (b, 0, 0)),
            pl.BlockSpec((1, N, 1), lambda b: (b, 0, 0)),
            full((D, HID)), full((1, HID)), full((1, HID)), full((1, HID)),
            full((HID, D)), full((1, D)), full((1, D)), full((1, D)),
            full((D, D)), full((1, D)),
            full((D, D)), full((1, D)),
            full((D, D)), full((1, D)),
            full((NH, DH, D)), full((1, D)),
        ],
        out_specs=pl.BlockSpec((1, N, D), lambda b: (b, 0, 0)),
        out_shape=jax.ShapeDtypeStruct((B, N, D), jnp.float32),
        compiler_params=pltpu.CompilerParams(
            dimension_semantics=("arbitrary",)),
    )(features, geo, lab,
      ag_w1, row2(ag_b1), row2(ag_g1), row2(ag_be1),
      ag_w2, row2(ag_b2), row2(ag_g2), row2(ag_be2),
      wq, row2(bq), wk, row2(bk), wv, row2(bv), wo_r, row2(bo))


# speedup vs baseline: 3.2064x; 1.0292x over previous
import jax
import jax.numpy as jnp
from jax.experimental import pallas as pl
from jax.experimental.pallas import tpu as pltpu

B, N, D, HID, S, NH = 2, 384, 768, 256, 64, 8
DH = D // NH
JB = 16
NJ = N // JB
EPS = 1e-5


def _ln(x, g, b):
    m = jnp.mean(x, axis=-1, keepdims=True)
    v = jnp.mean((x - m) ** 2, axis=-1, keepdims=True)
    return (x - m) * jax.lax.rsqrt(v + EPS) * g + b


def _geo_kernel(cj_ref, ci_ref, w1g_ref, u1_ref, m1_ref, g1c_ref, be1c_ref,
                w2t_ref, b2r_ref, b2c_ref, g2n_ref, be2c_ref, out_ref,
                rel_ref, hacc_ref, accmu_ref, accs_ref,
                m2b_ref, u2b_ref, c2b_ref, sc_ref, g1b_ref, be1b_ref):
    j_blk = pl.program_id(1)

    @pl.when(j_blk == 0)
    def _():
        g1b_ref[...] = jnp.broadcast_to(g1c_ref[...], (HID, N))
        be1b_ref[...] = jnp.broadcast_to(be1c_ref[...], (HID, N))
        w2t = w2t_ref[...]
        gram = jax.lax.dot_general(
            w2t, w2t, (((0,), (0,)), ((), ())),
            preferred_element_type=jnp.float32)
        m2b_ref[...] = (gram * (1.0 / D)).astype(jnp.bfloat16)
        u2b_ref[...] = jnp.mean(w2t, axis=0, keepdims=True).astype(jnp.bfloat16)
        b2r = b2r_ref[...]
        c2 = jnp.dot(b2r, w2t, preferred_element_type=jnp.float32) * (1.0 / D)
        c2b_ref[...] = c2.astype(jnp.bfloat16)
        sc_ref[0:1, 0:1] = jnp.mean(b2r, axis=1, keepdims=True)
        sc_ref[0:1, 1:2] = jnp.mean(b2r * b2r, axis=1, keepdims=True)
        hacc_ref[...] = jnp.zeros((HID, N), dtype=jnp.float32)
        accmu_ref[...] = jnp.zeros((1, N), dtype=jnp.float32)
        accs_ref[...] = jnp.zeros((1, N), dtype=jnp.float32)

    cix = ci_ref[0, 0:1, :]
    ciy = ci_ref[0, 1:2, :]
    ciz = ci_ref[0, 2:3, :]
    ones_row = jnp.ones((1, N), dtype=jnp.float32)
    mb2 = sc_ref[0:1, 0:1]
    s2 = sc_ref[0:1, 1:2]
    for t in range(JB):
        cjx = cj_ref[0, t:t + 1, 0:1]
        cjy = cj_ref[0, t:t + 1, 1:2]
        cjz = cj_ref[0, t:t + 1, 2:3]
        dx = cix - cjx
        dy = ciy - cjy
        dz = ciz - cjz
        dist = jnp.sqrt(dx * dx + dy * dy + dz * dz)
        inv = 1.0 / jnp.maximum(dist, 1e-12)
        rx = dx * inv
        ry = dy * inv
        rz = dz * inv
        rel_ref[0:1, :] = dist
        rel_ref[1:2, :] = rx
        rel_ref[2:3, :] = ry
        rel_ref[3:4, :] = rz
        rel_ref[4:5, :] = ry * rz
        rel_ref[5:6, :] = rx * ry
        rel_ref[6:7, :] = rx * rz
        rel_ref[7:8, :] = ones_row
        rel = rel_ref[...]
        mu1 = jnp.dot(u1_ref[...], rel, preferred_element_type=jnp.float32)
        q1 = jnp.dot(m1_ref[...], rel, preferred_element_type=jnp.float32)
        ms1 = jnp.sum(rel * q1, axis=0, keepdims=True)
        rs1 = jax.lax.rsqrt(jnp.maximum(ms1 - mu1 * mu1, 0.0) + EPS)
        rels = rel * rs1
        hp = jnp.dot(w1g_ref[...], rels, preferred_element_type=jnp.float32)
        adj = g1b_ref[...] * (mu1 * rs1) - be1b_ref[...]
        h = jax.nn.relu(hp - adj)
        hb = h.astype(jnp.bfloat16)
        mu2 = jnp.dot(u2b_ref[...], hb,
                      preferred_element_type=jnp.float32) + mb2
        q2 = jnp.dot(m2b_ref[...], hb, preferred_element_type=jnp.float32)
        c2h = jnp.dot(c2b_ref[...], hb, preferred_element_type=jnp.float32)
        ms2 = jnp.sum(h * q2, axis=0, keepdims=True) + 2.0 * c2h + s2
        rs2 = jax.lax.rsqrt(jnp.maximum(ms2 - mu2 * mu2, 0.0) + EPS)
        hacc_ref[...] += h * rs2
        accmu_ref[...] += mu2 * rs2
        accs_ref[...] += rs2

    @pl.when(j_blk == NJ - 1)
    def _():
        a = jnp.dot(w2t_ref[...], hacc_ref[...],
                    preferred_element_type=jnp.float32)
        a = a - accmu_ref[...] + b2c_ref[...] * accs_ref[...]
        out_ref[0] = a * g2n_ref[...] + be2c_ref[...]


def _agg_attn_kernel(feat_ref, geo_ref, lab_ref,
                     aw1_ref, ab1_ref, ag1_ref, abe1_ref,
                     aw2_ref, ab2_ref, ag2_ref, abe2_ref,
                     wq_ref, bq_ref, wk_ref, bk_ref, wv_ref, bv_ref,
                     wo_ref, bo_ref, out_ref):
    feat = feat_ref[0]
    geo_t = geo_ref[0]
    lab = lab_ref[0]
    iota_s = jax.lax.broadcasted_iota(jnp.int32, (N, S), 1)
    oh = (lab == iota_s).astype(jnp.float32)
    ones = jnp.ones((N, 1), dtype=jnp.float32)
    cn = (((0,), (0,)), ((), ()))
    cnt = jax.lax.dot_general(oh, ones, cn,
                              preferred_element_type=jnp.float32)
    sum_f = jax.lax.dot_general(oh, feat, cn,
                                preferred_element_type=jnp.float32)
    sum_g = jax.lax.dot_general(oh, geo_t, (((0,), (1,)), ((), ())),
                                preferred_element_type=jnp.float32)
    denom = 1.0 / jnp.maximum(cnt, 1.0)
    mean_f = sum_f * denom
    mean_g = sum_g * denom
    ah = jnp.dot(mean_f, aw1_ref[...], preferred_element_type=jnp.float32)
    ah = jax.nn.relu(_ln(ah + ab1_ref[0:1, :], ag1_ref[0:1, :], abe1_ref[0:1, :]))
    agg = jnp.dot(ah, aw2_ref[...], preferred_element_type=jnp.float32)
    agg = _ln(agg + ab2_ref[0:1, :], ag2_ref[0:1, :], abe2_ref[0:1, :])
    combined = agg + mean_g
    comb_g = jnp.dot(oh, combined, preferred_element_type=jnp.float32)
    cnt_pt = jnp.dot(oh, cnt, preferred_element_type=jnp.float32)
    valid = cnt_pt >= 2.0
    enh = jnp.where(valid, 0.7 * feat + 0.3 * comb_g, feat)
    scale = DH ** -0.5
    enh_b = enh.astype(jnp.bfloat16)
    geo_tb = geo_t.astype(jnp.bfloat16)
    tn = (((0,), (0,)), ((), ()))
    q = (jnp.dot(enh_b, wq_ref[...].astype(jnp.bfloat16),
                 preferred_element_type=jnp.float32)
         + bq_ref[...]).astype(jnp.bfloat16)
    k = (jax.lax.dot_general(geo_tb, wk_ref[...].astype(jnp.bfloat16), tn,
                             preferred_element_type=jnp.float32)
         + bk_ref[...]).astype(jnp.bfloat16)
    v = jax.lax.dot_general(geo_tb, wv_ref[...].astype(jnp.bfloat16), tn,
                            preferred_element_type=jnp.float32) + bv_ref[...]
    acc = jnp.zeros((N, D), dtype=jnp.float32)
    ct = (((1,), (1,)), ((), ()))
    for h in range(NH):
        qh = q[:, h * DH:(h + 1) * DH]
        kh = k[:, h * DH:(h + 1) * DH]
        vh = v[:, h * DH:(h + 1) * DH]
        s = jax.lax.dot_general(qh, kh, ct,
                                preferred_element_type=jnp.float32) * scale
        s = s - jnp.max(s, axis=-1, keepdims=True)
        p = jnp.exp(s)
        p = p * (1.0 / jnp.sum(p, axis=-1, keepdims=True))
        ov = jnp.dot(p, vh, preferred_element_type=jnp.float32)
        acc = acc + jnp.dot(ov, wo_ref[h], preferred_element_type=jnp.float32)
    out_ref[0] = enh + 0.5 * (acc + bo_ref[0:1, :])


def _geo_ctx(coordinates, ge_w1, ge_b1, ge_g1, ge_be1, ge_w2, ge_b2, ge_g2, ge_be2):
    coords_t = coordinates.transpose(0, 2, 1)
    w1ta = jnp.stack([ge_w1[0], ge_w1[1], ge_w1[2], ge_w1[3],
                      ge_w1[6], ge_w1[7], ge_w1[8], ge_b1], axis=1)
    u1 = jnp.mean(w1ta, axis=0, keepdims=True)
    m1 = (w1ta.T @ w1ta) / HID
    w1g = w1ta * ge_g1[:, None]
    w2t = ge_w2.T
    full = lambda shp: pl.BlockSpec(shp, lambda b, j: tuple(0 for _ in shp))
    geo_t = pl.pallas_call(
        _geo_kernel,
        grid=(B, NJ),
        in_specs=[
            pl.BlockSpec((1, JB, 3), lambda b, j: (b, j, 0)),
            pl.BlockSpec((1, 3, N), lambda b, j: (b, 0, 0)),
            full((HID, 8)), full((1, 8)), full((8, 8)),
            full((HID, 1)), full((HID, 1)),
            full((D, HID)), full((1, D)),
            full((D, 1)), full((D, 1)), full((D, 1)),
        ],
        out_specs=pl.BlockSpec((1, D, N), lambda b, j: (b, 0, 0)),
        out_shape=jax.ShapeDtypeStruct((B, D, N), jnp.float32),
        scratch_shapes=[pltpu.VMEM((8, N), jnp.float32),
                        pltpu.VMEM((HID, N), jnp.float32),
                        pltpu.VMEM((1, N), jnp.float32),
                        pltpu.VMEM((1, N), jnp.float32),
                        pltpu.VMEM((HID, HID), jnp.bfloat16),
                        pltpu.VMEM((1, HID), jnp.bfloat16),
                        pltpu.VMEM((1, HID), jnp.bfloat16),
                        pltpu.VMEM((1, 128), jnp.float32),
                        pltpu.VMEM((HID, N), jnp.float32),
                        pltpu.VMEM((HID, N), jnp.float32)],
        compiler_params=pltpu.CompilerParams(
            dimension_semantics=("arbitrary", "arbitrary")),
    )(coordinates, coords_t, w1g, u1, m1,
      ge_g1.reshape(HID, 1), ge_be1.reshape(HID, 1),
      w2t, ge_b2.reshape(1, D),
      ge_b2.reshape(D, 1), (ge_g2 / N).reshape(D, 1), ge_be2.reshape(D, 1))
    return geo_t


def kernel(coordinates, features, superpoint_labels,
           ge_w1, ge_b1, ge_g1, ge_be1, ge_w2, ge_b2, ge_g2, ge_be2,
           ag_w1, ag_b1, ag_g1, ag_be1, ag_w2, ag_b2, ag_g2, ag_be2,
           wq, bq, wk, bk, wv, bv, wo, bo):
    geo = _geo_ctx(coordinates, ge_w1, ge_b1, ge_g1, ge_be1,
                   ge_w2, ge_b2, ge_g2, ge_be2)
    lab = superpoint_labels.astype(jnp.int32).reshape(B, N, 1)
    row2 = lambda x: x.reshape(1, -1)
    wo_r = wo.reshape(NH, DH, D)
    full = lambda shp: pl.BlockSpec(shp, lambda b: tuple(0 for _ in shp))
    return pl.pallas_call(
        _agg_attn_kernel,
        grid=(B,),
        in_specs=[
            pl.BlockSpec((1, N, D), lambda b: (b, 0, 0)),
            pl.BlockSpec((1, D, N), lambda b: (b, 0, 0)),
            pl.BlockSpec((1, N, 1), lambda b: (b, 0, 0)),
            full((D, HID)), full((1, HID)), full((1, HID)), full((1, HID)),
            full((HID, D)), full((1, D)), full((1, D)), full((1, D)),
            full((D, D)), full((1, D)),
            full((D, D)), full((1, D)),
            full((D, D)), full((1, D)),
            full((NH, DH, D)), full((1, D)),
        ],
        out_specs=pl.BlockSpec((1, N, D), lambda b: (b, 0, 0)),
        out_shape=jax.ShapeDtypeStruct((B, N, D), jnp.float32),
        compiler_params=pltpu.CompilerParams(
            dimension_semantics=("arbitrary",)),
    )(features, geo, lab,
      ag_w1, row2(ag_b1), row2(ag_g1), row2(ag_be1),
      ag_w2, row2(ag_b2), row2(ag_g2), row2(ag_be2),
      wq, row2(bq), wk, row2(bk), wv, row2(bv), wo_r, row2(bo))
